# Initial kernel scaffold; baseline (speedup 1.0000x reference)
import functools

import jax
import jax.numpy as jnp
from jax import lax
from jax.experimental import pallas as pl
from jax.experimental.pallas import tpu as pltpu
from jax.experimental.pallas import tpu_sc as plsc

N = 10000
NFEAT = 128
NHID = 128
NCLASS = 64

NCORES = 2
NSUB = 16
NW = NCORES * NSUB
CH = 128
N_PAD = 10240

f32 = jnp.float32
i32 = jnp.int32


def _sc_degree(col, w, n_pad):
    e_pad = col.shape[0]
    ept = e_pad // NW
    nchunks = ept // CH
    rps = n_pad // NSUB

    mesh = plsc.VectorSubcoreMesh(core_axis_name="c", subcore_axis_name="s")

    @functools.partial(
        pl.kernel, mesh=mesh,
        out_type=jax.ShapeDtypeStruct((NCORES, n_pad, 16), f32),
        scratch_types=[
            pltpu.VMEM((CH,), i32),
            pltpu.VMEM((CH,), f32),
            pltpu.VMEM((CH, 16), f32),
            pltpu.VMEM_SHARED((n_pad, 16), f32),
        ],
    )
    def k(col_hbm, w_hbm, out_hbm, cbuf, wbuf, wrows, acc):
        cid = lax.axis_index("c")
        sid = lax.axis_index("s")
        wid = sid * NCORES + cid

        zero16 = jnp.zeros((16,), f32)

        @pl.loop(0, CH)
        def _(i):
            wrows.at[i][...] = zero16

        @pl.loop(0, rps, step=CH)
        def _(r):
            pltpu.sync_copy(wrows, acc.at[pl.ds(sid * rps + r, CH)])

        plsc.subcore_barrier()

        base0 = wid * ept

        @pl.loop(0, nchunks)
        def _(ci):
            base = base0 + ci * CH
            pltpu.sync_copy(col_hbm.at[pl.ds(base, CH)], cbuf)
            pltpu.sync_copy(w_hbm.at[pl.ds(base, CH)], wbuf)

            @pl.loop(0, CH)
            def _(e):
                we = wbuf[e]
                wrows.at[e][...] = jnp.full((16,), we, f32)

            pltpu.sync_copy(wrows, acc.at[cbuf], add=True)

        plsc.subcore_barrier()

        @pl.loop(0, rps, step=CH)
        def _(r):
            pltpu.sync_copy(acc.at[pl.ds(sid * rps + r, CH)],
                            out_hbm.at[cid, pl.ds(sid * rps + r, CH)])

    return k(col, w)


def _sc_propagate(xp, row, col, w, n_pad, feat):
    e_pad = row.shape[0]
    ept = e_pad // NW
    nchunks = ept // CH
    rps = n_pad // NSUB

    mesh = plsc.VectorSubcoreMesh(core_axis_name="c", subcore_axis_name="s")

    @functools.partial(
        pl.kernel, mesh=mesh,
        out_type=jax.ShapeDtypeStruct((NCORES, n_pad, feat), f32),
        scratch_types=[
            pltpu.VMEM((CH,), i32),
            pltpu.VMEM((CH,), i32),
            pltpu.VMEM((CH,), f32),
            pltpu.VMEM((CH, feat), f32),
            pltpu.VMEM_SHARED((n_pad, feat), f32),
        ],
    )
    def k(x_hbm, row_hbm, col_hbm, w_hbm, out_hbm,
          rbuf, cbuf, wbuf, rows, acc):
        cid = lax.axis_index("c")
        sid = lax.axis_index("s")
        wid = sid * NCORES + cid

        zero16 = jnp.zeros((16,), f32)

        @pl.loop(0, CH)
        def _(i):
            @pl.loop(0, feat, step=16)
            def _(j):
                rows.at[i, pl.ds(j, 16)][...] = zero16

        @pl.loop(0, rps, step=CH)
        def _(r):
            pltpu.sync_copy(rows, acc.at[pl.ds(sid * rps + r, CH)])

        plsc.subcore_barrier()

        base0 = wid * ept

        @pl.loop(0, nchunks)
        def _(ci):
            base = base0 + ci * CH
            pltpu.sync_copy(row_hbm.at[pl.ds(base, CH)], rbuf)
            pltpu.sync_copy(col_hbm.at[pl.ds(base, CH)], cbuf)
            pltpu.sync_copy(w_hbm.at[pl.ds(base, CH)], wbuf)
            pltpu.sync_copy(x_hbm.at[rbuf], rows)

            @pl.loop(0, CH)
            def _(e):
                we = wbuf[e]

                @pl.loop(0, feat, step=16)
                def _(j):
                    rows.at[e, pl.ds(j, 16)][...] = (
                        rows.at[e, pl.ds(j, 16)][...] * we)

            pltpu.sync_copy(rows, acc.at[cbuf], add=True)

        plsc.subcore_barrier()

        @pl.loop(0, rps, step=CH)
        def _(r):
            pltpu.sync_copy(acc.at[pl.ds(sid * rps + r, CH)],
                            out_hbm.at[cid, pl.ds(sid * rps + r, CH)])

    return k(xp, row, col, w)


BR = 256


def _dis_block(degw_ref):
    deg = degw_ref[0, :, 0] + degw_ref[1, :, 0] + 1.0
    return jnp.where(deg > 0, lax.rsqrt(jnp.maximum(deg, 1e-12)), 0.0)


def _tc_prescale(degw, x_pad):
    def body(degw_ref, x_ref, o_ref):
        dis = _dis_block(degw_ref)
        o_ref[...] = dis[:, None] * x_ref[...]

    return pl.pallas_call(
        body,
        grid=(N_PAD // BR,),
        in_specs=[
            pl.BlockSpec((NCORES, BR, 16), lambda i: (0, i, 0)),
            pl.BlockSpec((BR, NFEAT), lambda i: (i, 0)),
        ],
        out_specs=pl.BlockSpec((BR, NFEAT), lambda i: (i, 0)),
        out_shape=jax.ShapeDtypeStruct((N_PAD, NFEAT), f32),
    )(degw, x_pad)


def _tc_mid(degw, s1, xp, W1, b1, W2):
    def body(degw_ref, s1_ref, xp_ref, w1_ref, b1_ref, w2_ref, o_ref):
        dis = _dis_block(degw_ref)
        t = s1_ref[0] + s1_ref[1] + xp_ref[...]
        z = dis[:, None] * t
        h = lax.dot_general(z, w1_ref[...], (((1,), (1,)), ((), ())),
                            precision=lax.Precision.HIGHEST,
                            preferred_element_type=f32)
        h = jnp.maximum(h + b1_ref[...], 0.0)
        g = lax.dot_general(h, w2_ref[...], (((1,), (1,)), ((), ())),
                            precision=lax.Precision.HIGHEST,
                            preferred_element_type=f32)
        o_ref[...] = dis[:, None] * g

    return pl.pallas_call(
        body,
        grid=(N_PAD // BR,),
        in_specs=[
            pl.BlockSpec((NCORES, BR, 16), lambda i: (0, i, 0)),
            pl.BlockSpec((NCORES, BR, NHID), lambda i: (0, i, 0)),
            pl.BlockSpec((BR, NFEAT), lambda i: (i, 0)),
            pl.BlockSpec((NHID, NFEAT), lambda i: (0, 0)),
            pl.BlockSpec((1, NHID), lambda i: (0, 0)),
            pl.BlockSpec((NCLASS, NHID), lambda i: (0, 0)),
        ],
        out_specs=pl.BlockSpec((BR, NCLASS), lambda i: (i, 0)),
        out_shape=jax.ShapeDtypeStruct((N_PAD, NCLASS), f32),
    )(degw, s1, xp, W1, b1, W2)


def _tc_final(degw, s2, gp, b2):
    def body(degw_ref, s2_ref, gp_ref, b2_ref, o_ref):
        dis = _dis_block(degw_ref)
        z = dis[:, None] * (s2_ref[0] + s2_ref[1] + gp_ref[...]) + b2_ref[...]
        m = jnp.max(z, axis=1, keepdims=True)
        zm = z - m
        s = jnp.sum(jnp.exp(zm), axis=1, keepdims=True)
        o_ref[...] = zm - jnp.log(s)

    return pl.pallas_call(
        body,
        grid=(N_PAD // BR,),
        in_specs=[
            pl.BlockSpec((NCORES, BR, 16), lambda i: (0, i, 0)),
            pl.BlockSpec((NCORES, BR, NCLASS), lambda i: (0, i, 0)),
            pl.BlockSpec((BR, NCLASS), lambda i: (i, 0)),
            pl.BlockSpec((1, NCLASS), lambda i: (0, 0)),
        ],
        out_specs=pl.BlockSpec((BR, NCLASS), lambda i: (i, 0)),
        out_shape=jax.ShapeDtypeStruct((N_PAD, NCLASS), f32),
    )(degw, s2, gp, b2)


@jax.jit
def _run(features, edge_index, edge_weight, W1, b1, W2, b2):
    E = edge_index.shape[1]
    chunk = NW * CH
    e_pad = ((E + chunk - 1) // chunk) * chunk
    pad = e_pad - E

    row = jnp.pad(edge_index[0], (0, pad))
    col = jnp.pad(edge_index[1], (0, pad))
    w = jnp.pad(edge_weight, (0, pad))

    x_pad = jnp.pad(features, ((0, N_PAD - N), (0, 0)))

    degw = _sc_degree(col, w, N_PAD)
    xp = _tc_prescale(degw, x_pad)
    s1 = _sc_propagate(xp, row, col, w, N_PAD, NFEAT)
    gp = _tc_mid(degw, s1, xp, W1, b1.reshape(1, NHID), W2)
    s2 = _sc_propagate(gp, row, col, w, N_PAD, NCLASS)
    out = _tc_final(degw, s2, gp, b2.reshape(1, NCLASS))
    return out[:N]


def kernel(features, edge_index, edge_weight, W1, b1, W2, b2):
    return _run(features, edge_index, edge_weight, W1, b1, W2, b2)


# speedup vs baseline: 9.2529x; 9.2529x over previous
import dataclasses
import functools

import jax
import jax.numpy as jnp
from jax import lax
from jax.experimental import pallas as pl
from jax.experimental.pallas import tpu as pltpu
from jax.experimental.pallas import tpu_sc as plsc

N = 10000
NFEAT = 128
NHID = 128
NCLASS = 64

NCORES = 2
NSUB = 16
NW = NCORES * NSUB
CH = 128
N_PAD = 10240

f32 = jnp.float32
i32 = jnp.int32


def _sc_degree(col, w, n_pad):
    e_pad = col.shape[0]
    ept = e_pad // NW
    nchunks = ept // CH

    mesh = plsc.VectorSubcoreMesh(core_axis_name="c", subcore_axis_name="s")

    cp = pltpu.CompilerParams()
    if "needs_layout_passes" in pltpu.CompilerParams.__dataclass_fields__:
        cp = dataclasses.replace(cp, needs_layout_passes=False)

    @functools.partial(
        pl.kernel, mesh=mesh,
        out_type=jax.ShapeDtypeStruct((NW, n_pad), f32),
        compiler_params=cp,
        scratch_types=[
            pltpu.VMEM((CH,), i32),
            pltpu.VMEM((CH,), f32),
            pltpu.VMEM((n_pad,), f32),
        ],
    )
    def k(col_hbm, w_hbm, out_hbm, cbuf, wbuf, pdeg):
        cid = lax.axis_index("c")
        sid = lax.axis_index("s")
        wid = sid * NCORES + cid

        zero16 = jnp.zeros((16,), f32)

        @pl.loop(0, n_pad, step=16)
        def _(j):
            pdeg.at[pl.ds(j, 16)][...] = zero16

        base0 = wid * ept

        @pl.loop(0, nchunks)
        def _(ci):
            base = base0 + ci * CH
            pltpu.sync_copy(col_hbm.at[pl.ds(base, CH)], cbuf)
            pltpu.sync_copy(w_hbm.at[pl.ds(base, CH)], wbuf)

            @pl.loop(0, CH, step=16)
            def _(g):
                cv = cbuf[pl.ds(g, 16)]
                wv = wbuf[pl.ds(g, 16)]
                plsc.addupdate_scatter(pdeg, [cv], wv)

        pltpu.sync_copy(pdeg, out_hbm.at[wid])

    return k(col, w)


def _sc_propagate(xp, row, col, w, n_pad, feat):
    e_pad = row.shape[0]
    ept = e_pad // NW
    nchunks = ept // CH
    rps = n_pad // NSUB

    mesh = plsc.VectorSubcoreMesh(core_axis_name="c", subcore_axis_name="s")

    @functools.partial(
        pl.kernel, mesh=mesh,
        out_type=jax.ShapeDtypeStruct((NCORES, n_pad, feat), f32),
        scratch_types=[
            pltpu.VMEM((CH,), i32),
            pltpu.VMEM((CH,), i32),
            pltpu.VMEM((CH,), f32),
            pltpu.VMEM((CH, feat), f32),
            pltpu.VMEM_SHARED((n_pad, feat), f32),
        ],
    )
    def k(x_hbm, row_hbm, col_hbm, w_hbm, out_hbm,
          rbuf, cbuf, wbuf, rows, acc):
        cid = lax.axis_index("c")
        sid = lax.axis_index("s")
        wid = sid * NCORES + cid

        zero16 = jnp.zeros((16,), f32)

        @pl.loop(0, CH)
        def _(i):
            @pl.loop(0, feat, step=16)
            def _(j):
                rows.at[i, pl.ds(j, 16)][...] = zero16

        @pl.loop(0, rps, step=CH)
        def _(r):
            pltpu.sync_copy(rows, acc.at[pl.ds(sid * rps + r, CH)])

        plsc.subcore_barrier()

        base0 = wid * ept

        @pl.loop(0, nchunks)
        def _(ci):
            base = base0 + ci * CH
            pltpu.sync_copy(row_hbm.at[pl.ds(base, CH)], rbuf)
            pltpu.sync_copy(col_hbm.at[pl.ds(base, CH)], cbuf)
            pltpu.sync_copy(w_hbm.at[pl.ds(base, CH)], wbuf)
            pltpu.sync_copy(x_hbm.at[rbuf], rows)

            @pl.loop(0, CH, step=16)
            def _(g):
                wv = wbuf[pl.ds(g, 16)]
                for l in range(16):
                    we = wv[l]

                    @pl.loop(0, feat, step=16)
                    def _(j):
                        rows.at[g + l, pl.ds(j, 16)][...] = (
                            rows.at[g + l, pl.ds(j, 16)][...] * we)

            pltpu.sync_copy(rows, acc.at[cbuf], add=True)

        plsc.subcore_barrier()

        @pl.loop(0, rps, step=CH)
        def _(r):
            pltpu.sync_copy(acc.at[pl.ds(sid * rps + r, CH)],
                            out_hbm.at[cid, pl.ds(sid * rps + r, CH)])

    return k(xp, row, col, w)


BR = 1024
DIS_BLOCK = (BR, 1)


def _tc_dis(degw):
    def body(degw_ref, dis_ref):
        ones = jnp.ones((NW, 1), f32)
        deg = lax.dot_general(degw_ref[...], ones, (((0,), (0,)), ((), ())),
                              precision=lax.Precision.HIGHEST,
                              preferred_element_type=f32) + 1.0
        dis_ref[...] = jnp.where(
            deg > 0, lax.rsqrt(jnp.maximum(deg, 1e-12)), 0.0)

    return pl.pallas_call(
        body,
        grid=(N_PAD // BR,),
        in_specs=[pl.BlockSpec((NW, BR), lambda i: (0, i))],
        out_specs=pl.BlockSpec((BR, 1), lambda i: (i, 0)),
        out_shape=jax.ShapeDtypeStruct((N_PAD, 1), f32),
    )(degw)


def _tc_prescale(dis_img, x_pad):
    def body(dis_ref, x_ref, o_ref):
        o_ref[...] = dis_ref[...] * x_ref[...]

    return pl.pallas_call(
        body,
        grid=(N_PAD // BR,),
        in_specs=[
            pl.BlockSpec(DIS_BLOCK, lambda i: (i, 0)),
            pl.BlockSpec((BR, NFEAT), lambda i: (i, 0)),
        ],
        out_specs=pl.BlockSpec((BR, NFEAT), lambda i: (i, 0)),
        out_shape=jax.ShapeDtypeStruct((N_PAD, NFEAT), f32),
    )(dis_img, x_pad)


def _tc_mid(dis_img, s1, xp, W1, b1, W2):
    def body(dis_ref, s1_ref, xp_ref, w1_ref, b1_ref, w2_ref, o_ref):
        dis = dis_ref[...]
        t = s1_ref[0] + s1_ref[1] + xp_ref[...]
        z = dis * t
        h = lax.dot_general(z, w1_ref[...], (((1,), (1,)), ((), ())),
                            precision=lax.Precision.HIGHEST,
                            preferred_element_type=f32)
        h = jnp.maximum(h + b1_ref[...], 0.0)
        g = lax.dot_general(h, w2_ref[...], (((1,), (1,)), ((), ())),
                            precision=lax.Precision.HIGHEST,
                            preferred_element_type=f32)
        o_ref[...] = jnp.concatenate(
            [dis * g, jnp.zeros((BR, NHID - NCLASS), f32)], axis=1)

    return pl.pallas_call(
        body,
        grid=(N_PAD // BR,),
        in_specs=[
            pl.BlockSpec(DIS_BLOCK, lambda i: (i, 0)),
            pl.BlockSpec((NCORES, BR, NHID), lambda i: (0, i, 0)),
            pl.BlockSpec((BR, NFEAT), lambda i: (i, 0)),
            pl.BlockSpec((NHID, NFEAT), lambda i: (0, 0)),
            pl.BlockSpec((1, NHID), lambda i: (0, 0)),
            pl.BlockSpec((NCLASS, NHID), lambda i: (0, 0)),
        ],
        out_specs=pl.BlockSpec((BR, NHID), lambda i: (i, 0)),
        out_shape=jax.ShapeDtypeStruct((N_PAD, NHID), f32),
    )(dis_img, s1, xp, W1, b1, W2)


def _tc_final(dis_img, s2, gp, b2):
    def body(dis_ref, s2_ref, gp_ref, b2_ref, o_ref):
        dis = dis_ref[...]
        t = (s2_ref[0] + s2_ref[1] + gp_ref[...])[:, :NCLASS]
        z = dis * t + b2_ref[...]
        m = jnp.max(z, axis=1, keepdims=True)
        zm = z - m
        s = jnp.sum(jnp.exp(zm), axis=1, keepdims=True)
        o_ref[...] = zm - jnp.log(s)

    return pl.pallas_call(
        body,
        grid=(N_PAD // BR,),
        in_specs=[
            pl.BlockSpec(DIS_BLOCK, lambda i: (i, 0)),
            pl.BlockSpec((NCORES, BR, NHID), lambda i: (0, i, 0)),
            pl.BlockSpec((BR, NHID), lambda i: (i, 0)),
            pl.BlockSpec((1, NCLASS), lambda i: (0, 0)),
        ],
        out_specs=pl.BlockSpec((BR, NCLASS), lambda i: (i, 0)),
        out_shape=jax.ShapeDtypeStruct((N_PAD, NCLASS), f32),
    )(dis_img, s2, gp, b2)


@jax.jit
def _run(features, edge_index, edge_weight, W1, b1, W2, b2):
    E = edge_index.shape[1]
    chunk = NW * CH
    e_pad = ((E + chunk - 1) // chunk) * chunk
    pad = e_pad - E

    row = jnp.pad(edge_index[0], (0, pad))
    col = jnp.pad(edge_index[1], (0, pad))
    w = jnp.pad(edge_weight, (0, pad))

    x_pad = jnp.pad(features, ((0, N_PAD - N), (0, 0)))

    degw = _sc_degree(col, w, N_PAD)
    dis_img = _tc_dis(degw)
    xp = _tc_prescale(dis_img, x_pad)
    s1 = _sc_propagate(xp, row, col, w, N_PAD, NFEAT)
    gp = _tc_mid(dis_img, s1, xp, W1, b1.reshape(1, NHID), W2)
    s2 = _sc_propagate(gp, row, col, w, N_PAD, NHID)
    out = _tc_final(dis_img, s2, gp, b2.reshape(1, NCLASS))
    return out[:N]


def kernel(features, edge_index, edge_weight, W1, b1, W2, b2):
    return _run(features, edge_index, edge_weight, W1, b1, W2, b2)
